# baseline (speedup 1.0000x reference)
import functools
import math

import jax
import jax.numpy as jnp
from jax import lax
from jax.experimental import pallas as pl
from jax.experimental.pallas import tpu as pltpu
from jax.experimental.pallas import tpu_sc as plsc

_LS = 0.1
_V = 10000
_PAD = 0
_CONF = 1.0 - _LS
_SMOOTH = _LS / (_V - 2)
_C_ROW = (_V - 2) * _SMOOTH * math.log(_SMOOTH) + _CONF * math.log(_CONF)

_BLK = 256
_NC, _NS, _L = 2, 16, 16
_NW = _NC * _NS


def _rowsum_body(t_ref, x_ref, o_ref):
    x = x_ref[...]
    rowsum = jnp.sum(x, axis=1, keepdims=True)
    x0 = x[:, 0:1]
    t = t_ref[0]
    per_row = _C_ROW - _SMOOTH * (rowsum - x0)
    part = jnp.sum(jnp.where(t != _PAD, per_row, 0.0))

    @pl.when(pl.program_id(0) == 0)
    def _():
        o_ref[0, 0] = 0.0

    o_ref[0, 0] += part


def _sc_gather_body(n, v, table, tgt_hbm, out_hbm, idx_t, widx, vals, accv, sem):
    b_w = n // _NW
    wid = lax.axis_index("s") * _NC + lax.axis_index("c")
    base = wid * b_w
    pltpu.sync_copy(tgt_hbm.at[pl.ds(base, b_w)], idx_t)
    ar = lax.iota(jnp.int32, 16)
    acc = jnp.zeros((16,), jnp.float32)
    for r in range(b_w // 128):
        off = r * 128
        for g in range(8):
            t = idx_t[pl.ds(off + g * 16, 16)]
            widx[pl.ds(g * 16, 16)] = (base + off + g * 16 + ar) * v + t
        pltpu.async_copy(table.at[widx], vals, sem).wait()
        for g in range(8):
            t = idx_t[pl.ds(off + g * 16, 16)]
            vv = vals[pl.ds(g * 16, 16)]
            acc = acc + jnp.where(t != _PAD, vv, 0.0)
    accv[...] = acc
    pltpu.sync_copy(accv, out_hbm.at[wid])


def kernel(output, target):
    n, v = output.shape
    nblk = n // _BLK
    b_w = n // _NW
    tgt = target.astype(jnp.int32)
    t3 = tgt.reshape(nblk, _BLK, 1)

    table = output.reshape(n * v)
    mesh = plsc.VectorSubcoreMesh(
        core_axis_name="c", subcore_axis_name="s", num_cores=_NC, num_subcores=_NS
    )
    sc_parts = pl.kernel(
        functools.partial(_sc_gather_body, n, v),
        out_type=jax.ShapeDtypeStruct((_NW, _L), jnp.float32),
        mesh=mesh,
        scratch_types=[
            pltpu.VMEM((b_w,), jnp.int32),
            pltpu.VMEM((128,), jnp.int32),
            pltpu.VMEM((128,), jnp.float32),
            pltpu.VMEM((_L,), jnp.float32),
            pltpu.SemaphoreType.DMA,
        ],
    )(table, tgt)

    tc_out = pl.pallas_call(
        _rowsum_body,
        grid=(nblk,),
        in_specs=[
            pl.BlockSpec((1, _BLK, 1), lambda i: (i, 0, 0)),
            pl.BlockSpec((_BLK, v), lambda i: (i, 0)),
        ],
        out_specs=pl.BlockSpec(memory_space=pltpu.SMEM),
        out_shape=jax.ShapeDtypeStruct((1, 1), jnp.float32),
    )(t3, output)

    return tc_out[0, 0] - (_CONF - _SMOOTH) * jnp.sum(sc_parts)


# speedup vs baseline: 6.2313x; 6.2313x over previous
import math

import jax
import jax.numpy as jnp
from jax.experimental import pallas as pl
from jax.experimental.pallas import tpu as pltpu

_LS = 0.1
_V = 10000
_PAD = 0
_CONF = 1.0 - _LS
_SMOOTH = _LS / (_V - 2)
_C_ROW = (_V - 2) * _SMOOTH * math.log(_SMOOTH) + _CONF * math.log(_CONF)

_VB = 200


def _loss_body(t_ref, x_ref, o_ref):
    b = pl.program_id(0)
    x = x_ref[...]
    t = t_ref[...]
    colsum = jnp.sum(x, axis=0, keepdims=True)
    rows = jax.lax.broadcasted_iota(jnp.int32, x.shape, 0) + b * _VB
    tval = jnp.sum(jnp.where(rows == t, x, 0.0), axis=0, keepdims=True)
    contrib = -_SMOOTH * colsum - (_CONF - _SMOOTH) * tval
    head = jnp.where(b == 0, _C_ROW + _SMOOTH * x[0:1, :], 0.0)
    part = jnp.sum(jnp.where(t != _PAD, contrib + head, 0.0))

    @pl.when(b == 0)
    def _():
        o_ref[0, 0] = 0.0

    o_ref[0, 0] += part


def kernel(output, target):
    n, v = output.shape
    xt = output.T
    t2 = target.astype(jnp.int32).reshape(1, n)
    out = pl.pallas_call(
        _loss_body,
        grid=(v // _VB,),
        in_specs=[
            pl.BlockSpec((1, n), lambda b: (0, 0)),
            pl.BlockSpec((_VB, n), lambda b: (b, 0)),
        ],
        out_specs=pl.BlockSpec(memory_space=pltpu.SMEM),
        out_shape=jax.ShapeDtypeStruct((1, 1), jnp.float32),
    )(t2, xt)
    return out[0, 0]
